# Initial kernel scaffold; baseline (speedup 1.0000x reference)
import functools

import jax
import jax.numpy as jnp
from jax.experimental import pallas as pl
from jax.experimental.pallas import tpu as pltpu
from jax.experimental.pallas import tpu_sc as plsc

_WINDOW = 1024


def _sc_gather(table, idx):
    m = idx.shape[1]
    d = table.shape[1]
    mesh = plsc.VectorSubcoreMesh(core_axis_name="core", subcore_axis_name="subcore")

    @functools.partial(
        pl.kernel,
        out_type=jax.ShapeDtypeStruct((m, d), table.dtype),
        mesh=mesh,
    )
    def k(table_hbm, idx_hbm, out_hbm):
        def body(i_vmem, o_vmem):
            pltpu.sync_copy(table_hbm.at[i_vmem.at[0]], o_vmem)

        pltpu.emit_pipeline(
            body,
            grid=(m // _WINDOW,),
            in_specs=[pl.BlockSpec((1, _WINDOW), index_map=lambda i: (0, i))],
            out_specs=[pl.BlockSpec((_WINDOW, d), index_map=lambda i: (i, 0))],
            core_axis_name=("core", "subcore"),
            dimension_semantics=(pltpu.PARALLEL,),
        )(idx_hbm, out_hbm)

    return k(table, idx)


def kernel(lat, lon, sog, cog, W_lat, W_lon, W_sog, W_cog):
    b, l = lat.shape
    n = b * l
    d = W_lat.shape[1]

    table = jnp.concatenate([W_lat, W_lon, W_sog, W_cog], axis=0)
    o1 = W_lat.shape[0]
    o2 = o1 + W_lon.shape[0]
    o3 = o2 + W_sog.shape[0]
    idx = jnp.stack(
        [
            lat.reshape(n).astype(jnp.int32),
            lon.reshape(n).astype(jnp.int32) + o1,
            sog.reshape(n).astype(jnp.int32) + o2,
            cog.reshape(n).astype(jnp.int32) + o3,
        ],
        axis=-1,
    ).reshape(1, 4 * n)

    out = _sc_gather(table, idx)
    return out.reshape(b, l, 4 * d)


# speedup vs baseline: 8.3241x; 8.3241x over previous
import functools

import jax
import jax.numpy as jnp
from jax.experimental import pallas as pl
from jax.experimental.pallas import tpu as pltpu
from jax.experimental.pallas import tpu_sc as plsc

_WINDOW = 1024


def _sc_gather(table, idx):
    m = idx.shape[1]
    d = table.shape[1]
    mesh = plsc.VectorSubcoreMesh(core_axis_name="core", subcore_axis_name="subcore")

    @functools.partial(
        pl.kernel,
        out_type=jax.ShapeDtypeStruct((m, d), table.dtype),
        mesh=mesh,
        compiler_params=pltpu.CompilerParams(use_tc_tiling_on_sc=False),
    )
    def k(table_hbm, idx_hbm, out_hbm):
        def body(i_vmem, o_vmem):
            pltpu.sync_copy(table_hbm.at[i_vmem.at[0]], o_vmem)

        pltpu.emit_pipeline(
            body,
            grid=(m // _WINDOW,),
            in_specs=[pl.BlockSpec((1, _WINDOW), index_map=lambda i: (0, i))],
            out_specs=[pl.BlockSpec((_WINDOW, d), index_map=lambda i: (i, 0))],
            core_axis_name=("core", "subcore"),
            dimension_semantics=(pltpu.PARALLEL,),
        )(idx_hbm, out_hbm)

    return k(table, idx)


def kernel(lat, lon, sog, cog, W_lat, W_lon, W_sog, W_cog):
    b, l = lat.shape
    n = b * l
    d = W_lat.shape[1]

    table = jnp.concatenate([W_lat, W_lon, W_sog, W_cog], axis=0)
    o1 = W_lat.shape[0]
    o2 = o1 + W_lon.shape[0]
    o3 = o2 + W_sog.shape[0]
    idx = jnp.stack(
        [
            lat.reshape(n).astype(jnp.int32),
            lon.reshape(n).astype(jnp.int32) + o1,
            sog.reshape(n).astype(jnp.int32) + o2,
            cog.reshape(n).astype(jnp.int32) + o3,
        ],
        axis=-1,
    ).reshape(1, 4 * n)

    out = _sc_gather(table, idx)
    return out.reshape(b, l, 4 * d)


# speedup vs baseline: 8.3492x; 1.0030x over previous
import functools

import jax
import jax.numpy as jnp
from jax.experimental import pallas as pl
from jax.experimental.pallas import tpu as pltpu
from jax.experimental.pallas import tpu_sc as plsc

_WINDOW = 1600


def _sc_gather(table, idx):
    m = idx.shape[1]
    d = table.shape[1]
    mesh = plsc.VectorSubcoreMesh(core_axis_name="core", subcore_axis_name="subcore")

    @functools.partial(
        pl.kernel,
        out_type=jax.ShapeDtypeStruct((m, d), table.dtype),
        mesh=mesh,
        compiler_params=pltpu.CompilerParams(use_tc_tiling_on_sc=False),
    )
    def k(table_hbm, idx_hbm, out_hbm):
        def body(i_vmem, o_vmem):
            pltpu.sync_copy(table_hbm.at[i_vmem.at[0]], o_vmem)

        pltpu.emit_pipeline(
            body,
            grid=(m // _WINDOW,),
            in_specs=[pl.BlockSpec((1, _WINDOW), index_map=lambda i: (0, i))],
            out_specs=[pl.BlockSpec((_WINDOW, d), index_map=lambda i: (i, 0))],
            core_axis_name=("core", "subcore"),
            dimension_semantics=(pltpu.PARALLEL,),
        )(idx_hbm, out_hbm)

    return k(table, idx)


def kernel(lat, lon, sog, cog, W_lat, W_lon, W_sog, W_cog):
    b, l = lat.shape
    n = b * l
    d = W_lat.shape[1]

    table = jnp.concatenate([W_lat, W_lon, W_sog, W_cog], axis=0)
    o1 = W_lat.shape[0]
    o2 = o1 + W_lon.shape[0]
    o3 = o2 + W_sog.shape[0]
    idx = jnp.stack(
        [
            lat.reshape(n).astype(jnp.int32),
            lon.reshape(n).astype(jnp.int32) + o1,
            sog.reshape(n).astype(jnp.int32) + o2,
            cog.reshape(n).astype(jnp.int32) + o3,
        ],
        axis=-1,
    ).reshape(1, 4 * n)

    out = _sc_gather(table, idx)
    return out.reshape(b, l, 4 * d)


# speedup vs baseline: 9.1987x; 1.1018x over previous
import functools

import jax
import jax.numpy as jnp
from jax.experimental import pallas as pl
from jax.experimental.pallas import tpu as pltpu
from jax.experimental.pallas import tpu_sc as plsc

_WINDOW = 1600


def _sc_gather(table, idx):
    m = idx.shape[1]
    d = table.shape[1]
    mesh = plsc.VectorSubcoreMesh(core_axis_name="core", subcore_axis_name="subcore")

    @functools.partial(
        pl.kernel,
        out_type=jax.ShapeDtypeStruct((m, d), table.dtype),
        mesh=mesh,
        compiler_params=pltpu.CompilerParams(use_tc_tiling_on_sc=False),
    )
    def k(table_hbm, idx_hbm, out_hbm):
        def body(i_vmem, o_vmem):
            pltpu.sync_copy(table_hbm.at[i_vmem.at[0]], o_vmem)

        pltpu.emit_pipeline(
            body,
            grid=(m // _WINDOW,),
            in_specs=[pl.BlockSpec((1, _WINDOW), index_map=lambda i: (0, i))],
            out_specs=[pl.BlockSpec((_WINDOW, d), index_map=lambda i: (i, 0))],
            core_axis_name=("core", "subcore"),
            dimension_semantics=(pltpu.PARALLEL,),
        )(idx_hbm, out_hbm)

    return k(table, idx)


_REPL = 32


def kernel(lat, lon, sog, cog, W_lat, W_lon, W_sog, W_cog):
    b, l = lat.shape
    n = b * l
    d = W_lat.shape[1]

    sog_rep = jnp.tile(W_sog, (_REPL, 1))
    cog_rep = jnp.tile(W_cog, (_REPL, 1))
    table = jnp.concatenate([W_lat, W_lon, sog_rep, cog_rep], axis=0)
    o1 = W_lat.shape[0]
    o2 = o1 + W_lon.shape[0]
    o3 = o2 + sog_rep.shape[0]
    replica = jnp.arange(n, dtype=jnp.int32) % _REPL
    idx = jnp.stack(
        [
            lat.reshape(n).astype(jnp.int32),
            lon.reshape(n).astype(jnp.int32) + o1,
            sog.reshape(n).astype(jnp.int32) + (o2 + replica * W_sog.shape[0]),
            cog.reshape(n).astype(jnp.int32) + (o3 + replica * W_cog.shape[0]),
        ],
        axis=-1,
    ).reshape(1, 4 * n)

    out = _sc_gather(table, idx)
    return out.reshape(b, l, 4 * d)


# speedup vs baseline: 34.2039x; 3.7183x over previous
import functools

import jax
import jax.numpy as jnp
from jax.experimental import pallas as pl
from jax.experimental.pallas import tpu as pltpu
from jax.experimental.pallas import tpu_sc as plsc

_WINDOW = 400
_REPL = 32


def _sc_gather4(table, idx):
    m = idx.shape[1]
    d = table.shape[1]
    mesh = plsc.VectorSubcoreMesh(core_axis_name="core", subcore_axis_name="subcore")

    @functools.partial(
        pl.kernel,
        out_type=jax.ShapeDtypeStruct((m, 4 * d), table.dtype),
        mesh=mesh,
        compiler_params=pltpu.CompilerParams(use_tc_tiling_on_sc=False),
    )
    def k(table_hbm, idx_hbm, out_hbm):
        def body(i0, i1, i2, i3, o0, o1, o2, o3):
            for i_vmem, o_vmem in zip((i0, i1, i2, i3), (o0, o1, o2, o3)):
                pltpu.sync_copy(table_hbm.at[i_vmem.at[0]], o_vmem)

        pltpu.emit_pipeline(
            body,
            grid=(m // _WINDOW,),
            in_specs=[
                pl.BlockSpec((1, _WINDOW), index_map=lambda i, f=f: (f, i))
                for f in range(4)
            ],
            out_specs=[
                pl.BlockSpec((_WINDOW, d), index_map=lambda i, f=f: (i, f))
                for f in range(4)
            ],
            core_axis_name=("core", "subcore"),
            dimension_semantics=(pltpu.PARALLEL,),
        )(idx_hbm, idx_hbm, idx_hbm, idx_hbm, out_hbm, out_hbm, out_hbm, out_hbm)

    return k(table, idx)


def kernel(lat, lon, sog, cog, W_lat, W_lon, W_sog, W_cog):
    b, l = lat.shape
    n = b * l
    d = W_lat.shape[1]

    sog_rep = jnp.tile(W_sog, (_REPL, 1))
    cog_rep = jnp.tile(W_cog, (_REPL, 1))
    table = jnp.concatenate([W_lat, W_lon, sog_rep, cog_rep], axis=0)
    o1 = W_lat.shape[0]
    o2 = o1 + W_lon.shape[0]
    o3 = o2 + sog_rep.shape[0]
    replica = jnp.arange(n, dtype=jnp.int32) % _REPL
    idx = jnp.stack(
        [
            lat.reshape(n).astype(jnp.int32),
            lon.reshape(n).astype(jnp.int32) + o1,
            sog.reshape(n).astype(jnp.int32) + (o2 + replica * W_sog.shape[0]),
            cog.reshape(n).astype(jnp.int32) + (o3 + replica * W_cog.shape[0]),
        ],
        axis=0,
    )

    out = _sc_gather4(table, idx)
    return out.reshape(b, l, 4 * d)


# speedup vs baseline: 43.9341x; 1.2845x over previous
import functools

import jax
import jax.numpy as jnp
from jax.experimental import pallas as pl
from jax.experimental.pallas import tpu as pltpu
from jax.experimental.pallas import tpu_sc as plsc

_WINDOW = 400
_REPL = 32


def _sc_gather4(table, idx):
    m = idx.shape[1]
    d = table.shape[1]
    mesh = plsc.VectorSubcoreMesh(core_axis_name="core", subcore_axis_name="subcore")

    @functools.partial(
        pl.kernel,
        out_type=jax.ShapeDtypeStruct((m, 4 * d), table.dtype),
        mesh=mesh,
        scratch_types=[pltpu.SemaphoreType.DMA],
        compiler_params=pltpu.CompilerParams(use_tc_tiling_on_sc=False),
    )
    def k(table_hbm, idx_hbm, out_hbm, sem):
        def body(i0, i1, i2, i3, o0, o1, o2, o3):
            copies = [
                pltpu.async_copy(table_hbm.at[i_vmem.at[0]], o_vmem, sem)
                for i_vmem, o_vmem in zip((i0, i1, i2, i3), (o0, o1, o2, o3))
            ]
            for c in copies:
                c.wait()

        pltpu.emit_pipeline(
            body,
            grid=(m // _WINDOW,),
            in_specs=[
                pl.BlockSpec((1, _WINDOW), index_map=lambda i, f=f: (f, i))
                for f in range(4)
            ],
            out_specs=[
                pl.BlockSpec((_WINDOW, d), index_map=lambda i, f=f: (i, f))
                for f in range(4)
            ],
            core_axis_name=("core", "subcore"),
            dimension_semantics=(pltpu.PARALLEL,),
        )(idx_hbm, idx_hbm, idx_hbm, idx_hbm, out_hbm, out_hbm, out_hbm, out_hbm)

    return k(table, idx)


def kernel(lat, lon, sog, cog, W_lat, W_lon, W_sog, W_cog):
    b, l = lat.shape
    n = b * l
    d = W_lat.shape[1]

    sog_rep = jnp.tile(W_sog, (_REPL, 1))
    cog_rep = jnp.tile(W_cog, (_REPL, 1))
    table = jnp.concatenate([W_lat, W_lon, sog_rep, cog_rep], axis=0)
    o1 = W_lat.shape[0]
    o2 = o1 + W_lon.shape[0]
    o3 = o2 + sog_rep.shape[0]
    replica = jnp.arange(n, dtype=jnp.int32) % _REPL
    idx = jnp.stack(
        [
            lat.reshape(n).astype(jnp.int32),
            lon.reshape(n).astype(jnp.int32) + o1,
            sog.reshape(n).astype(jnp.int32) + (o2 + replica * W_sog.shape[0]),
            cog.reshape(n).astype(jnp.int32) + (o3 + replica * W_cog.shape[0]),
        ],
        axis=0,
    )

    out = _sc_gather4(table, idx)
    return out.reshape(b, l, 4 * d)
